# baseline (speedup 1.0000x reference)
import functools

import jax
import jax.numpy as jnp
from jax import lax
from jax.experimental import pallas as pl
from jax.experimental.pallas import tpu as pltpu
from jax.experimental.pallas import tpu_sc as plsc

N_NODES = 10000
D = 128
E_EDGES = 320000

NCORES = 2
NSUB = 16
NTILES = NCORES * NSUB

NP = 10240
ROWS_PER_TILE = NP // NSUB

CHUNK = 128
NCHUNK = 82
EPAD = NTILES * NCHUNK * CHUNK

BLK = 2048
INIT_ROWS = 128

_mesh = plsc.VectorSubcoreMesh(core_axis_name="c", subcore_axis_name="s")


@functools.partial(
    pl.kernel,
    out_type=jax.ShapeDtypeStruct((NCORES, NP), jnp.float32),
    mesh=_mesh,
    scratch_types=[
        pltpu.VMEM((NCHUNK, CHUNK), jnp.int32),
        pltpu.VMEM((CHUNK,), jnp.float32),
        pltpu.VMEM((ROWS_PER_TILE,), jnp.float32),
        pltpu.VMEM_SHARED((NP,), jnp.float32),
    ],
)
def _degree_kernel(dst_hbm, ones_hbm, zeros_hbm, out_hbm, idst, ones_v, zeros_v,
                   acc):
    cid = lax.axis_index("c")
    sid = lax.axis_index("s")
    wid = sid * NCORES + cid
    base = sid * ROWS_PER_TILE
    pltpu.sync_copy(dst_hbm.at[wid], idst)
    pltpu.sync_copy(ones_hbm, ones_v)
    pltpu.sync_copy(zeros_hbm.at[pl.ds(base, ROWS_PER_TILE)], zeros_v)

    pltpu.sync_copy(zeros_v, acc.at[pl.ds(base, ROWS_PER_TILE)])
    plsc.subcore_barrier()

    def body(j, carry):
        pltpu.sync_copy(ones_v, acc.at[idst.at[j]], add=True)
        return carry

    lax.fori_loop(0, NCHUNK, body, 0)
    plsc.subcore_barrier()
    pltpu.sync_copy(acc.at[pl.ds(base, ROWS_PER_TILE)],
                    out_hbm.at[cid, pl.ds(base, ROWS_PER_TILE)])


@functools.partial(
    pl.kernel,
    out_type=jax.ShapeDtypeStruct((NCORES, NP, D), jnp.float32),
    mesh=_mesh,
    scratch_types=[
        pltpu.VMEM((2, CHUNK), jnp.int32),
        pltpu.VMEM((2, CHUNK), jnp.int32),
        pltpu.VMEM_SHARED((NP, D), jnp.float32),
        pltpu.SemaphoreType.DMA,
        pltpu.SemaphoreType.DMA,
    ],
)
def _scatter_kernel(y_hbm, eidx_hbm, z_hbm, out_hbm, ibuf0, ibuf1,
                    acc, sem0, sem1):
    cid = lax.axis_index("c")
    sid = lax.axis_index("s")
    wid = sid * NCORES + cid
    base = sid * ROWS_PER_TILE

    pltpu.sync_copy(z_hbm, acc.at[pl.ds(base, ROWS_PER_TILE)])

    def run(rows0, rows1):
        plsc.subcore_barrier()

        pltpu.sync_copy(eidx_hbm.at[wid, 0], ibuf0)
        pltpu.async_copy(y_hbm.at[ibuf0.at[0]], rows0, sem0)

        def body(p, carry):
            j0 = 2 * p
            pltpu.sync_copy(eidx_hbm.at[wid, j0 + 1], ibuf1)
            pltpu.make_async_copy(y_hbm.at[ibuf0.at[0]], rows0, sem0).wait()
            pltpu.async_copy(y_hbm.at[ibuf1.at[0]], rows1, sem1)
            pltpu.sync_copy(rows0, acc.at[ibuf0.at[1]], add=True)

            @pl.when(j0 + 2 < NCHUNK)
            def _():
                pltpu.sync_copy(eidx_hbm.at[wid, j0 + 2], ibuf0)
                pltpu.async_copy(y_hbm.at[ibuf0.at[0]], rows0, sem0)

            pltpu.make_async_copy(y_hbm.at[ibuf1.at[0]], rows1,
                                  sem1).wait()
            pltpu.sync_copy(rows1, acc.at[ibuf1.at[1]], add=True)
            return carry

        lax.fori_loop(0, NCHUNK // 2, body, 0)

    pl.run_scoped(run, pltpu.VMEM((CHUNK, D), jnp.float32),
                  pltpu.VMEM((CHUNK, D), jnp.float32))
    plsc.subcore_barrier()
    pltpu.sync_copy(acc.at[pl.ds(base, ROWS_PER_TILE)],
                    out_hbm.at[cid, pl.ds(base, ROWS_PER_TILE)])


def _tc_first(d0, d1, x_p, W):
    def kfn(d0_ref, d1_ref, x_ref, w_ref, y_ref):
        s = lax.rsqrt(d0_ref[...] + d1_ref[...])
        h = jnp.dot(x_ref[...], w_ref[...], preferred_element_type=jnp.float32)
        y_ref[...] = h * s

    return pl.pallas_call(
        kfn,
        grid=(NP // BLK,),
        in_specs=[
            pl.BlockSpec((BLK, 1), lambda i: (i, 0)),
            pl.BlockSpec((BLK, 1), lambda i: (i, 0)),
            pl.BlockSpec((BLK, D), lambda i: (i, 0)),
            pl.BlockSpec((D, D), lambda i: (0, 0)),
        ],
        out_specs=pl.BlockSpec((BLK, D), lambda i: (i, 0)),
        out_shape=jax.ShapeDtypeStruct((NP, D), jnp.float32),
    )(d0, d1, x_p, W)


def _tc_mid(d0, d1, a0, a1, b, W):
    def kfn(d0_ref, d1_ref, a0_ref, a1_ref, b_ref, w_ref, y_ref):
        s = lax.rsqrt(d0_ref[...] + d1_ref[...])
        z = (a0_ref[...] + a1_ref[...]) * s + b_ref[...]
        z = jnp.maximum(z, 0.0)
        h = jnp.dot(z, w_ref[...], preferred_element_type=jnp.float32)
        y_ref[...] = h * s

    return pl.pallas_call(
        kfn,
        grid=(NP // BLK,),
        in_specs=[
            pl.BlockSpec((BLK, 1), lambda i: (i, 0)),
            pl.BlockSpec((BLK, 1), lambda i: (i, 0)),
            pl.BlockSpec((BLK, D), lambda i: (i, 0)),
            pl.BlockSpec((BLK, D), lambda i: (i, 0)),
            pl.BlockSpec((1, D), lambda i: (0, 0)),
            pl.BlockSpec((D, D), lambda i: (0, 0)),
        ],
        out_specs=pl.BlockSpec((BLK, D), lambda i: (i, 0)),
        out_shape=jax.ShapeDtypeStruct((NP, D), jnp.float32),
    )(d0, d1, a0, a1, b, W)


def _tc_last(d0, d1, a0, a1, b):
    def kfn(d0_ref, d1_ref, a0_ref, a1_ref, b_ref, o_ref):
        s = lax.rsqrt(d0_ref[...] + d1_ref[...])
        o_ref[...] = (a0_ref[...] + a1_ref[...]) * s + b_ref[...]

    return pl.pallas_call(
        kfn,
        grid=(NP // BLK,),
        in_specs=[
            pl.BlockSpec((BLK, 1), lambda i: (i, 0)),
            pl.BlockSpec((BLK, 1), lambda i: (i, 0)),
            pl.BlockSpec((BLK, D), lambda i: (i, 0)),
            pl.BlockSpec((BLK, D), lambda i: (i, 0)),
            pl.BlockSpec((1, D), lambda i: (0, 0)),
        ],
        out_specs=pl.BlockSpec((BLK, D), lambda i: (i, 0)),
        out_shape=jax.ShapeDtypeStruct((NP, D), jnp.float32),
    )(d0, d1, a0, a1, b)


def kernel(x, edge_index, W1, b1, W2, b2):
    n = x.shape[0]
    loops = jnp.arange(NP, dtype=jnp.int32)
    pad_e = EPAD - E_EDGES - NP
    src = jnp.concatenate([edge_index[0], loops,
                           jnp.full((pad_e,), n, jnp.int32)])
    dst = jnp.concatenate([edge_index[1], loops,
                           jnp.full((pad_e,), n, jnp.int32)])
    src3 = src.reshape(NTILES, NCHUNK, CHUNK)
    dst3 = dst.reshape(NTILES, NCHUNK, CHUNK)
    eidx = jnp.stack([src3, dst3], axis=2)

    x_p = jnp.pad(x, ((0, NP - n), (0, 0)))
    ones_c = jnp.ones((CHUNK,), jnp.float32)
    zeros_np = jnp.zeros((NP,), jnp.float32)
    zeros_nd = jnp.zeros((ROWS_PER_TILE, D), jnp.float32)

    degp = _degree_kernel(dst3, ones_c, zeros_np)
    d0 = degp[0].reshape(NP, 1)
    d1 = degp[1].reshape(NP, 1)

    b1r = b1.reshape(1, D)
    b2r = b2.reshape(1, D)

    y1 = _tc_first(d0, d1, x_p, W1)
    acc1 = _scatter_kernel(y1, eidx, zeros_nd)
    y2 = _tc_mid(d0, d1, acc1[0], acc1[1], b1r, W2)
    acc2 = _scatter_kernel(y2, eidx, zeros_nd)
    out = _tc_last(d0, d1, acc2[0], acc2[1], b2r)
    return out[:n]
